# baseline (speedup 1.0000x reference)
import functools

import jax
import jax.numpy as jnp
from jax import lax
from jax.experimental import pallas as pl
from jax.experimental.pallas import tpu as pltpu
from jax.experimental.pallas import tpu_sc as plsc

L = 16
NC = 2
NS = 16
NW = NC * NS


def _build(B, D, n_ctx, pre_len, suf_len, V):
    SEQ = 1 + pre_len + n_ctx + suf_len + 1
    COMB = 1 + pre_len
    HEAD = COMB
    TAIL = suf_len + 1
    DT = D // 128
    BPW = B // NW
    CHUNK = 8
    NCHUNK = BPW // CHUNK
    NG = B // CHUNK

    mesh = plsc.VectorSubcoreMesh(core_axis_name="c", subcore_axis_name="s")

    @functools.partial(
        pl.kernel,
        out_type=jax.ShapeDtypeStruct((SEQ, NG, DT, CHUNK, 128), jnp.float32),
        mesh=mesh,
        scratch_types=[
            pltpu.VMEM((BPW,), jnp.int32),
            pltpu.VMEM((DT, n_ctx, 128), jnp.float32),
            pltpu.VMEM((HEAD + TAIL, D), jnp.float32),
            pltpu.VMEM((CHUNK, DT, n_ctx, 128), jnp.float32),
            pltpu.VMEM((CHUNK, DT, n_ctx, 128), jnp.float32),
            pltpu.VMEM((n_ctx, 1, DT, CHUNK, 128), jnp.float32),
            pltpu.VMEM((n_ctx, 1, DT, CHUNK, 128), jnp.float32),
            pltpu.VMEM((HEAD, 1, DT, CHUNK, 128), jnp.float32),
            pltpu.VMEM((TAIL, 1, DT, CHUNK, 128), jnp.float32),
            pltpu.SemaphoreType.DMA,
            pltpu.SemaphoreType.DMA,
            pltpu.SemaphoreType.DMA,
            pltpu.SemaphoreType.DMA,
        ],
    )
    def k(pids_hbm, ctx_hbm, table_hbm, pre_hbm, suf_hbm, sos_hbm, eos_hbm,
          out_hbm, idx_v, ctx_v, const_v, gath0, gath1, comb0, comb1,
          head_v, tail_v, sg0, sg1, so0, so1):
        gath = (gath0, gath1)
        comb = (comb0, comb1)
        sg = (sg0, sg1)
        so = (so0, so1)
        wid = lax.axis_index("s") * NC + lax.axis_index("c")
        base = wid * BPW
        gbase = wid * NCHUNK
        pltpu.sync_copy(pids_hbm.at[pl.ds(base, BPW)], idx_v)
        pltpu.sync_copy(ctx_hbm, ctx_v)
        start_gather_c0 = pltpu.async_copy(
            table_hbm.at[idx_v.at[pl.ds(0, CHUNK)]], gath[0], sg[0])
        start_gather_c1 = pltpu.async_copy(
            table_hbm.at[idx_v.at[pl.ds(CHUNK, CHUNK)]], gath[1], sg[1])

        pltpu.sync_copy(sos_hbm, const_v.at[pl.ds(0, 1)])
        pltpu.sync_copy(pre_hbm, const_v.at[pl.ds(1, pre_len)])
        pltpu.sync_copy(suf_hbm, const_v.at[pl.ds(HEAD, suf_len)])
        pltpu.sync_copy(eos_hbm, const_v.at[pl.ds(HEAD + suf_len, 1)])

        @pl.loop(0, HEAD)
        def _fh(p):
            @pl.loop(0, DT)
            def _ft_(t):
                @pl.loop(0, 128 // L)
                def _fj(j):
                    s = j * L
                    row = const_v[p, pl.ds(t * 128 + s, L)]

                    @pl.loop(0, CHUNK, unroll=CHUNK)
                    def _fr(r):
                        head_v[p, 0, t, r, pl.ds(s, L)] = row

        @pl.loop(0, TAIL)
        def _ftl(p):
            @pl.loop(0, DT)
            def _ft_(t):
                @pl.loop(0, 128 // L)
                def _fj(j):
                    s = j * L
                    row = const_v[HEAD + p, pl.ds(t * 128 + s, L)]

                    @pl.loop(0, CHUNK, unroll=CHUNK)
                    def _fr(r):
                        tail_v[p, 0, t, r, pl.ds(s, L)] = row

        def start_gather(c):
            b = c % 2
            pltpu.async_copy(
                table_hbm.at[idx_v.at[pl.ds(c * CHUNK, CHUNK)]],
                gath[b], sg[b])

        def wait_gather(c):
            b = c % 2
            pltpu.make_async_copy(
                table_hbm.at[pl.ds(0, CHUNK)], gath[b], sg[b]).wait()

        def start_out(c):
            b = c % 2
            grp = pl.ds(gbase + c, 1)
            pltpu.async_copy(comb[b],
                             out_hbm.at[pl.ds(COMB, n_ctx), grp], so[b])
            pltpu.async_copy(head_v, out_hbm.at[pl.ds(0, HEAD), grp], so[b])
            pltpu.async_copy(tail_v,
                             out_hbm.at[pl.ds(COMB + n_ctx, TAIL), grp],
                             so[b])

        def wait_out(c):
            b = c % 2
            grp = pl.ds(0, 1)
            pltpu.make_async_copy(
                comb[b], out_hbm.at[pl.ds(COMB, n_ctx), grp], so[b]).wait()
            pltpu.make_async_copy(
                head_v, out_hbm.at[pl.ds(0, HEAD), grp], so[b]).wait()
            pltpu.make_async_copy(
                tail_v, out_hbm.at[pl.ds(COMB + n_ctx, TAIL), grp],
                so[b]).wait()

        def compute(c):
            b = c % 2
            gb = gath[b]
            cb = comb[b]

            @pl.loop(0, n_ctx)
            def _cc(cc):
                @pl.loop(0, DT)
                def _ct(t):
                    @pl.loop(0, 128 // L)
                    def _cj(j):
                        s = j * L
                        cvec = ctx_v[t, cc, pl.ds(s, L)]

                        @pl.loop(0, CHUNK, unroll=CHUNK)
                        def _cr(r):
                            cb[cc, 0, t, r, pl.ds(s, L)] = (
                                gb[r, t, cc, pl.ds(s, L)] + cvec)

        for c in range(NCHUNK):
            wait_gather(c)
            if c >= 2:
                wait_out(c - 2)
            compute(c)
            start_out(c)
            if c + 2 < NCHUNK:
                start_gather(c + 2)
        wait_out(NCHUNK - 2)
        wait_out(NCHUNK - 1)

    return k


def kernel(pids, ctx, class_ctx, prefix_emb, suffix_emb, sos_emb, eos_emb):
    B = pids.shape[0]
    n_ctx, D = ctx.shape
    V = class_ctx.shape[0]
    pre_len = prefix_emb.shape[0]
    suf_len = suffix_emb.shape[0]
    SEQ = 1 + pre_len + n_ctx + suf_len + 1
    DT = D // 128

    table = class_ctx.reshape(V, n_ctx, DT, 128).transpose(0, 2, 1, 3)
    ctx_t = ctx.reshape(n_ctx, DT, 128).transpose(1, 0, 2)

    k = _build(B, D, n_ctx, pre_len, suf_len, V)
    out5 = k(pids.astype(jnp.int32), ctx_t, table, prefix_emb, suffix_emb,
             sos_emb, eos_emb)
    return out5.transpose(1, 3, 0, 2, 4).reshape(B, SEQ, D)


# speedup vs baseline: 1.0061x; 1.0061x over previous
import functools

import jax
import jax.numpy as jnp
from jax import lax
from jax.experimental import pallas as pl
from jax.experimental.pallas import tpu as pltpu
from jax.experimental.pallas import tpu_sc as plsc

L = 16
NC = 2
NS = 16
NW = NC * NS


def _build(B, D, n_ctx, pre_len, suf_len, V):
    SEQ = 1 + pre_len + n_ctx + suf_len + 1
    COMB = 1 + pre_len
    HEAD = COMB
    TAIL = suf_len + 1
    BPW = B // NW
    CHUNK = 8
    NCHUNK = BPW // CHUNK

    mesh = plsc.VectorSubcoreMesh(core_axis_name="c", subcore_axis_name="s")

    @functools.partial(
        pl.kernel,
        out_type=jax.ShapeDtypeStruct((SEQ, B, D), jnp.float32),
        mesh=mesh,
        scratch_types=[
            pltpu.VMEM((BPW,), jnp.int32),
            pltpu.VMEM((n_ctx, D), jnp.float32),
            pltpu.VMEM((HEAD + TAIL, D), jnp.float32),
            pltpu.VMEM((CHUNK, n_ctx, D), jnp.float32),
            pltpu.VMEM((CHUNK, n_ctx, D), jnp.float32),
            pltpu.VMEM((n_ctx, CHUNK, D), jnp.float32),
            pltpu.VMEM((n_ctx, CHUNK, D), jnp.float32),
            pltpu.VMEM((HEAD, CHUNK, D), jnp.float32),
            pltpu.VMEM((TAIL, CHUNK, D), jnp.float32),
            pltpu.VMEM_SHARED((HEAD, CHUNK, D), jnp.float32),
            pltpu.VMEM_SHARED((TAIL, CHUNK, D), jnp.float32),
            pltpu.SemaphoreType.DMA,
            pltpu.SemaphoreType.DMA,
            pltpu.SemaphoreType.DMA,
            pltpu.SemaphoreType.DMA,
        ],
    )
    def k(pids_hbm, ctx_hbm, table_hbm, pre_hbm, suf_hbm, sos_hbm, eos_hbm,
          out_hbm, idx_v, ctx_v, const_v, gath0, gath1, comb0, comb1,
          head_v, tail_v, sh_head, sh_tail, sg0, sg1, so0, so1):
        gath = (gath0, gath1)
        comb = (comb0, comb1)
        sg = (sg0, sg1)
        so = (so0, so1)
        wid = lax.axis_index("s") * NC + lax.axis_index("c")
        base = wid * BPW
        pltpu.sync_copy(pids_hbm.at[pl.ds(base, BPW)], idx_v)
        pltpu.sync_copy(ctx_hbm, ctx_v)
        pltpu.sync_copy(sos_hbm, const_v.at[pl.ds(0, 1)])
        pltpu.sync_copy(pre_hbm, const_v.at[pl.ds(1, pre_len)])
        pltpu.sync_copy(suf_hbm, const_v.at[pl.ds(HEAD, suf_len)])
        pltpu.sync_copy(eos_hbm, const_v.at[pl.ds(HEAD + suf_len, 1)])

        @pl.loop(0, HEAD)
        def _fh(p):
            @pl.loop(0, D // L)
            def _fj(j):
                s = j * L
                row = const_v[p, pl.ds(s, L)]

                @pl.loop(0, CHUNK, unroll=CHUNK)
                def _fr(r):
                    head_v[p, r, pl.ds(s, L)] = row

        @pl.loop(0, TAIL)
        def _ft(p):
            @pl.loop(0, D // L)
            def _fj(j):
                s = j * L
                row = const_v[HEAD + p, pl.ds(s, L)]

                @pl.loop(0, CHUNK, unroll=CHUNK)
                def _fr(r):
                    tail_v[p, r, pl.ds(s, L)] = row

        @pl.when(lax.axis_index("s") == 0)
        def _fill_shared():
            pltpu.sync_copy(head_v, sh_head)
            pltpu.sync_copy(tail_v, sh_tail)

        plsc.subcore_barrier()

        def start_gather(c):
            b = c % 2
            pltpu.async_copy(
                table_hbm.at[idx_v.at[pl.ds(c * CHUNK, CHUNK)]],
                gath[b], sg[b])

        def wait_gather(c):
            b = c % 2
            pltpu.make_async_copy(
                table_hbm.at[pl.ds(0, CHUNK)], gath[b], sg[b]).wait()

        def start_out(c):
            b = c % 2
            rows = pl.ds(base + c * CHUNK, CHUNK)
            pltpu.async_copy(comb[b],
                             out_hbm.at[pl.ds(COMB, n_ctx), rows, :], so[b])
            pltpu.async_copy(sh_head, out_hbm.at[pl.ds(0, HEAD), rows, :],
                             so[b])
            pltpu.async_copy(sh_tail,
                             out_hbm.at[pl.ds(COMB + n_ctx, TAIL), rows, :],
                             so[b])

        def wait_out(c):
            b = c % 2
            rows = pl.ds(0, CHUNK)
            pltpu.make_async_copy(
                comb[b], out_hbm.at[pl.ds(COMB, n_ctx), rows, :],
                so[b]).wait()
            pltpu.make_async_copy(
                sh_head, out_hbm.at[pl.ds(0, HEAD), rows, :], so[b]).wait()
            pltpu.make_async_copy(
                sh_tail, out_hbm.at[pl.ds(COMB + n_ctx, TAIL), rows, :],
                so[b]).wait()

        def compute(c):
            b = c % 2
            gb = gath[b]
            cb = comb[b]

            @pl.loop(0, n_ctx)
            def _cc(cc):
                @pl.loop(0, D // L)
                def _cj(j):
                    s = j * L
                    cvec = ctx_v[cc, pl.ds(s, L)]

                    @pl.loop(0, CHUNK, unroll=CHUNK)
                    def _cr(r):
                        cb[cc, r, pl.ds(s, L)] = gb[r, cc, pl.ds(s, L)] + cvec

        start_gather(0)
        start_gather(1)
        for c in range(NCHUNK):
            wait_gather(c)
            if c >= 2:
                wait_out(c - 2)
            compute(c)
            start_out(c)
            if c + 2 < NCHUNK:
                start_gather(c + 2)
        wait_out(NCHUNK - 2)
        wait_out(NCHUNK - 1)

    return k


def kernel(pids, ctx, class_ctx, prefix_emb, suffix_emb, sos_emb, eos_emb):
    B = pids.shape[0]
    n_ctx, D = ctx.shape
    V = class_ctx.shape[0]
    pre_len = prefix_emb.shape[0]
    suf_len = suffix_emb.shape[0]

    k = _build(B, D, n_ctx, pre_len, suf_len, V)
    out = k(pids.astype(jnp.int32), ctx, class_ctx, prefix_emb, suffix_emb,
            sos_emb, eos_emb)
    return out.transpose(1, 0, 2)


# speedup vs baseline: 1.0072x; 1.0011x over previous
import functools

import jax
import jax.numpy as jnp
from jax import lax
from jax.experimental import pallas as pl
from jax.experimental.pallas import tpu as pltpu
from jax.experimental.pallas import tpu_sc as plsc

L = 16
NC = 2
NS = 16
NW = NC * NS


def _build(B, D, n_ctx, pre_len, suf_len, V):
    SEQ = 1 + pre_len + n_ctx + suf_len + 1
    COMB = 1 + pre_len
    HEAD = COMB
    TAIL = suf_len + 1
    BPW = B // NW
    CHUNK = 8
    NCHUNK = BPW // CHUNK

    mesh = plsc.VectorSubcoreMesh(core_axis_name="c", subcore_axis_name="s")

    @functools.partial(
        pl.kernel,
        out_type=jax.ShapeDtypeStruct((SEQ, B, D), jnp.float32),
        mesh=mesh,
        scratch_types=[
            pltpu.VMEM((BPW,), jnp.int32),
            pltpu.VMEM((n_ctx, D), jnp.float32),
            pltpu.VMEM((HEAD + TAIL, D), jnp.float32),
            pltpu.VMEM((CHUNK, n_ctx, D), jnp.float32),
            pltpu.VMEM((CHUNK, n_ctx, D), jnp.float32),
            pltpu.VMEM((n_ctx, CHUNK, D), jnp.float32),
            pltpu.VMEM((n_ctx, CHUNK, D), jnp.float32),
            pltpu.VMEM((HEAD, CHUNK, D), jnp.float32),
            pltpu.VMEM((TAIL, CHUNK, D), jnp.float32),
            pltpu.VMEM_SHARED((HEAD, CHUNK, D), jnp.float32),
            pltpu.VMEM_SHARED((TAIL, CHUNK, D), jnp.float32),
            pltpu.SemaphoreType.DMA,
            pltpu.SemaphoreType.DMA,
            pltpu.SemaphoreType.DMA,
            pltpu.SemaphoreType.DMA,
            pltpu.SemaphoreType.DMA,
        ],
    )
    def k(pids_hbm, ctx_hbm, table_hbm, pre_hbm, suf_hbm, sos_hbm, eos_hbm,
          out_hbm, idx_v, ctx_v, const_v, gath0, gath1, comb0, comb1,
          head_v, tail_v, sh_head, sh_tail, sg0, sg1, so0, so1, sconst):
        gath = (gath0, gath1)
        comb = (comb0, comb1)
        sg = (sg0, sg1)
        so = (so0, so1)
        wid = lax.axis_index("s") * NC + lax.axis_index("c")
        base = wid * BPW
        pltpu.sync_copy(pids_hbm.at[pl.ds(base, BPW)], idx_v)
        pltpu.sync_copy(ctx_hbm, ctx_v)
        pltpu.sync_copy(sos_hbm, const_v.at[pl.ds(0, 1)])
        pltpu.sync_copy(pre_hbm, const_v.at[pl.ds(1, pre_len)])
        pltpu.sync_copy(suf_hbm, const_v.at[pl.ds(HEAD, suf_len)])
        pltpu.sync_copy(eos_hbm, const_v.at[pl.ds(HEAD + suf_len, 1)])

        @pl.loop(0, HEAD)
        def _fh(p):
            @pl.loop(0, D // L)
            def _fj(j):
                s = j * L
                row = const_v[p, pl.ds(s, L)]

                @pl.loop(0, CHUNK, unroll=CHUNK)
                def _fr(r):
                    head_v[p, r, pl.ds(s, L)] = row

        @pl.loop(0, TAIL)
        def _ft(p):
            @pl.loop(0, D // L)
            def _fj(j):
                s = j * L
                row = const_v[HEAD + p, pl.ds(s, L)]

                @pl.loop(0, CHUNK, unroll=CHUNK)
                def _fr(r):
                    tail_v[p, r, pl.ds(s, L)] = row

        @pl.when(lax.axis_index("s") == 0)
        def _fill_shared():
            pltpu.sync_copy(head_v, sh_head)
            pltpu.sync_copy(tail_v, sh_tail)

        plsc.subcore_barrier()

        def start_gather(c):
            b = c % 2
            pltpu.async_copy(
                table_hbm.at[idx_v.at[pl.ds(c * CHUNK, CHUNK)]],
                gath[b], sg[b])

        def wait_gather(c):
            b = c % 2
            pltpu.make_async_copy(
                table_hbm.at[pl.ds(0, CHUNK)], gath[b], sg[b]).wait()

        def start_out(c):
            b = c % 2
            rows = pl.ds(base + c * CHUNK, CHUNK)
            pltpu.async_copy(comb[b],
                             out_hbm.at[pl.ds(COMB, n_ctx), rows, :], so[b])
            pltpu.async_copy(sh_head, out_hbm.at[pl.ds(0, HEAD), rows, :],
                             sconst)
            pltpu.async_copy(sh_tail,
                             out_hbm.at[pl.ds(COMB + n_ctx, TAIL), rows, :],
                             sconst)

        def wait_out(c):
            b = c % 2
            rows = pl.ds(0, CHUNK)
            pltpu.make_async_copy(
                comb[b], out_hbm.at[pl.ds(COMB, n_ctx), rows, :],
                so[b]).wait()

        def wait_const(c):
            rows = pl.ds(0, CHUNK)
            pltpu.make_async_copy(
                sh_head, out_hbm.at[pl.ds(0, HEAD), rows, :], sconst).wait()
            pltpu.make_async_copy(
                sh_tail, out_hbm.at[pl.ds(COMB + n_ctx, TAIL), rows, :],
                sconst).wait()

        def compute(c):
            b = c % 2
            gb = gath[b]
            cb = comb[b]

            @pl.loop(0, n_ctx)
            def _cc(cc):
                @pl.loop(0, D // L)
                def _cj(j):
                    s = j * L
                    cvec = ctx_v[cc, pl.ds(s, L)]

                    @pl.loop(0, CHUNK, unroll=CHUNK)
                    def _cr(r):
                        cb[cc, r, pl.ds(s, L)] = gb[r, cc, pl.ds(s, L)] + cvec

        start_gather(0)
        start_gather(1)
        for c in range(NCHUNK):
            wait_gather(c)
            if c >= 2:
                wait_out(c - 2)
            compute(c)
            if c + 2 < NCHUNK:
                start_gather(c + 2)
            start_out(c)
        wait_out(NCHUNK - 2)
        wait_out(NCHUNK - 1)
        for c in range(NCHUNK):
            wait_const(c)

    return k


def kernel(pids, ctx, class_ctx, prefix_emb, suffix_emb, sos_emb, eos_emb):
    B = pids.shape[0]
    n_ctx, D = ctx.shape
    V = class_ctx.shape[0]
    pre_len = prefix_emb.shape[0]
    suf_len = suffix_emb.shape[0]

    k = _build(B, D, n_ctx, pre_len, suf_len, V)
    out = k(pids.astype(jnp.int32), ctx, class_ctx, prefix_emb, suffix_emb,
            sos_emb, eos_emb)
    return out.transpose(1, 0, 2)


# speedup vs baseline: 1.4423x; 1.4319x over previous
import functools

import jax
import jax.numpy as jnp
from jax import lax
from jax.experimental import pallas as pl
from jax.experimental.pallas import tpu as pltpu
from jax.experimental.pallas import tpu_sc as plsc

L = 16
NC = 2
NS = 16
NW = NC * NS


def _build(B, D, n_ctx, pre_len, suf_len, V):
    SEQ = 1 + pre_len + n_ctx + suf_len + 1
    COMB = 1 + pre_len
    HEAD = COMB
    TAIL = suf_len + 1
    BPW = B // NW
    CHUNK = 8
    NCHUNK = BPW // CHUNK

    mesh = plsc.VectorSubcoreMesh(core_axis_name="c", subcore_axis_name="s")

    @functools.partial(
        pl.kernel,
        out_type=jax.ShapeDtypeStruct((SEQ, B, D), jnp.float32),
        mesh=mesh,
        scratch_types=[
            pltpu.VMEM((BPW,), jnp.int32),
            pltpu.VMEM((n_ctx, D), jnp.float32),
            pltpu.VMEM((HEAD + TAIL, D), jnp.float32),
            pltpu.VMEM((CHUNK, n_ctx, D), jnp.float32),
            pltpu.VMEM((CHUNK, n_ctx, D), jnp.float32),
            pltpu.VMEM((n_ctx, CHUNK, D), jnp.float32),
            pltpu.VMEM((n_ctx, CHUNK, D), jnp.float32),
            pltpu.VMEM((HEAD, CHUNK, D), jnp.float32),
            pltpu.VMEM((TAIL, CHUNK, D), jnp.float32),
            pltpu.VMEM_SHARED((HEAD, CHUNK, D), jnp.float32),
            pltpu.VMEM_SHARED((TAIL, CHUNK, D), jnp.float32),
            pltpu.SemaphoreType.DMA,
            pltpu.SemaphoreType.DMA,
            pltpu.SemaphoreType.DMA,
            pltpu.SemaphoreType.DMA,
            pltpu.SemaphoreType.DMA,
        ],
    )
    def k(pids_hbm, ctx_hbm, table_hbm, pre_hbm, suf_hbm, sos_hbm, eos_hbm,
          out_hbm, idx_v, ctx_v, const_v, gath0, gath1, comb0, comb1,
          head_v, tail_v, sh_head, sh_tail, sg0, sg1, so0, so1, sconst):
        gath = (gath0, gath1)
        comb = (comb0, comb1)
        sg = (sg0, sg1)
        so = (so0, so1)
        wid = lax.axis_index("s") * NC + lax.axis_index("c")
        base = wid * BPW
        pltpu.sync_copy(pids_hbm.at[pl.ds(base, BPW)], idx_v)
        pltpu.sync_copy(ctx_hbm, ctx_v)
        pltpu.sync_copy(sos_hbm, const_v.at[pl.ds(0, 1)])
        pltpu.sync_copy(pre_hbm, const_v.at[pl.ds(1, pre_len)])
        pltpu.sync_copy(suf_hbm, const_v.at[pl.ds(HEAD, suf_len)])
        pltpu.sync_copy(eos_hbm, const_v.at[pl.ds(HEAD + suf_len, 1)])

        @pl.loop(0, HEAD)
        def _fh(p):
            @pl.loop(0, D // L)
            def _fj(j):
                s = j * L
                row = const_v[p, pl.ds(s, L)]

                @pl.loop(0, CHUNK, unroll=CHUNK)
                def _fr(r):
                    head_v[p, r, pl.ds(s, L)] = row

        @pl.loop(0, TAIL)
        def _ft(p):
            @pl.loop(0, D // L)
            def _fj(j):
                s = j * L
                row = const_v[HEAD + p, pl.ds(s, L)]

                @pl.loop(0, CHUNK, unroll=CHUNK)
                def _fr(r):
                    tail_v[p, r, pl.ds(s, L)] = row

        @pl.when(lax.axis_index("s") == 0)
        def _fill_shared():
            pltpu.sync_copy(head_v, sh_head)
            pltpu.sync_copy(tail_v, sh_tail)

        plsc.subcore_barrier()

        def start_gather(c, b):
            pltpu.async_copy(
                table_hbm.at[idx_v.at[pl.ds(c * CHUNK, CHUNK)]],
                gath[b], sg[b])

        def wait_gather(b):
            pltpu.make_async_copy(
                table_hbm.at[pl.ds(0, CHUNK)], gath[b], sg[b]).wait()

        def start_out(c, b):
            rows = pl.ds(base + c * CHUNK, CHUNK)
            pltpu.async_copy(comb[b],
                             out_hbm.at[pl.ds(COMB, n_ctx), rows, :], so[b])
            pltpu.async_copy(sh_head, out_hbm.at[pl.ds(0, HEAD), rows, :],
                             sconst)
            pltpu.async_copy(sh_tail,
                             out_hbm.at[pl.ds(COMB + n_ctx, TAIL), rows, :],
                             sconst)

        def wait_out(b):
            rows = pl.ds(0, CHUNK)
            pltpu.make_async_copy(
                comb[b], out_hbm.at[pl.ds(COMB, n_ctx), rows, :],
                so[b]).wait()

        def wait_const():
            rows = pl.ds(0, CHUNK)
            pltpu.make_async_copy(
                sh_head, out_hbm.at[pl.ds(0, HEAD), rows, :], sconst).wait()
            pltpu.make_async_copy(
                sh_tail, out_hbm.at[pl.ds(COMB + n_ctx, TAIL), rows, :],
                sconst).wait()

        def compute(b):
            gb = gath[b]
            cb = comb[b]
            for cc in range(n_ctx):
                @plsc.parallel_loop(0, D // L, unroll=4)
                def _cj(j):
                    s = j * L
                    cvec = ctx_v[cc, pl.ds(s, L)]
                    for r in range(CHUNK):
                        cb[cc, r, pl.ds(s, L)] = gb[r, cc, pl.ds(s, L)] + cvec

        start_gather(0, 0)
        start_gather(1, 1)

        def pair_body(i, _):
            for par in range(2):
                c = 2 * i + par
                wait_gather(par)

                @pl.when(i > 0)
                def _wo():
                    wait_out(par)

                compute(par)

                @pl.when(c + 2 < NCHUNK)
                def _sg():
                    start_gather(c + 2, par)

                start_out(c, par)
            return ()

        lax.fori_loop(0, NCHUNK // 2, pair_body, (), unroll=False)
        wait_out(0)
        wait_out(1)
        for _ in range(NCHUNK):
            wait_const()

    return k


def kernel(pids, ctx, class_ctx, prefix_emb, suffix_emb, sos_emb, eos_emb):
    B = pids.shape[0]
    n_ctx, D = ctx.shape
    V = class_ctx.shape[0]
    pre_len = prefix_emb.shape[0]
    suf_len = suffix_emb.shape[0]

    k = _build(B, D, n_ctx, pre_len, suf_len, V)
    out = k(pids.astype(jnp.int32), ctx, class_ctx, prefix_emb, suffix_emb,
            sos_emb, eos_emb)
    return out.transpose(1, 0, 2)


# speedup vs baseline: 1.4477x; 1.0038x over previous
import functools

import jax
import jax.numpy as jnp
from jax import lax
from jax.experimental import pallas as pl
from jax.experimental.pallas import tpu as pltpu
from jax.experimental.pallas import tpu_sc as plsc

L = 16
NC = 2
NS = 16
NW = NC * NS


def _build(B, D, n_ctx, pre_len, suf_len, V):
    SEQ = 1 + pre_len + n_ctx + suf_len + 1
    COMB = 1 + pre_len
    HEAD = COMB
    TAIL = suf_len + 1
    BPW = B // NW
    CHUNK = 8
    NCHUNK = BPW // CHUNK

    mesh = plsc.VectorSubcoreMesh(core_axis_name="c", subcore_axis_name="s")

    @functools.partial(
        pl.kernel,
        out_type=jax.ShapeDtypeStruct((SEQ, B, D), jnp.float32),
        mesh=mesh,
        scratch_types=[
            pltpu.VMEM((BPW,), jnp.int32),
            pltpu.VMEM((n_ctx, D), jnp.float32),
            pltpu.VMEM((HEAD + TAIL, D), jnp.float32),
            pltpu.VMEM((CHUNK, n_ctx, D), jnp.float32),
            pltpu.VMEM((CHUNK, n_ctx, D), jnp.float32),
            pltpu.VMEM((n_ctx, CHUNK, D), jnp.float32),
            pltpu.VMEM((n_ctx, CHUNK, D), jnp.float32),
            pltpu.VMEM((HEAD, CHUNK, D), jnp.float32),
            pltpu.VMEM((TAIL, CHUNK, D), jnp.float32),
            pltpu.VMEM_SHARED((HEAD, CHUNK, D), jnp.float32),
            pltpu.VMEM_SHARED((TAIL, CHUNK, D), jnp.float32),
            pltpu.SemaphoreType.DMA,
            pltpu.SemaphoreType.DMA,
            pltpu.SemaphoreType.DMA,
            pltpu.SemaphoreType.DMA,
            pltpu.SemaphoreType.DMA,
        ],
    )
    def k(pids_hbm, ctx_hbm, table_hbm, pre_hbm, suf_hbm, sos_hbm, eos_hbm,
          out_hbm, idx_v, ctx_v, const_v, gath0, gath1, comb0, comb1,
          head_v, tail_v, sh_head, sh_tail, sg0, sg1, so0, so1, sconst):
        gath = (gath0, gath1)
        comb = (comb0, comb1)
        sg = (sg0, sg1)
        so = (so0, so1)
        wid = lax.axis_index("s") * NC + lax.axis_index("c")
        base = wid * BPW
        pltpu.sync_copy(pids_hbm.at[pl.ds(base, BPW)], idx_v)
        pltpu.sync_copy(ctx_hbm, ctx_v)
        pltpu.sync_copy(sos_hbm, const_v.at[pl.ds(0, 1)])
        pltpu.sync_copy(pre_hbm, const_v.at[pl.ds(1, pre_len)])
        pltpu.sync_copy(suf_hbm, const_v.at[pl.ds(HEAD, suf_len)])
        pltpu.sync_copy(eos_hbm, const_v.at[pl.ds(HEAD + suf_len, 1)])

        @pl.loop(0, HEAD)
        def _fh(p):
            @pl.loop(0, D // L)
            def _fj(j):
                s = j * L
                row = const_v[p, pl.ds(s, L)]

                @pl.loop(0, CHUNK, unroll=CHUNK)
                def _fr(r):
                    head_v[p, r, pl.ds(s, L)] = row

        @pl.loop(0, TAIL)
        def _ft(p):
            @pl.loop(0, D // L)
            def _fj(j):
                s = j * L
                row = const_v[HEAD + p, pl.ds(s, L)]

                @pl.loop(0, CHUNK, unroll=CHUNK)
                def _fr(r):
                    tail_v[p, r, pl.ds(s, L)] = row

        @pl.when(lax.axis_index("s") == 0)
        def _fill_shared():
            pltpu.sync_copy(head_v, sh_head)
            pltpu.sync_copy(tail_v, sh_tail)

        plsc.subcore_barrier()

        def start_gather(c, b):
            pltpu.async_copy(
                table_hbm.at[idx_v.at[pl.ds(c * CHUNK, CHUNK)]],
                gath[b], sg[b])

        def wait_gather(b):
            pltpu.make_async_copy(
                table_hbm.at[pl.ds(0, CHUNK)], gath[b], sg[b]).wait()

        def start_out(c, b):
            rows = pl.ds(base + c * CHUNK, CHUNK)
            pltpu.async_copy(comb[b],
                             out_hbm.at[pl.ds(COMB, n_ctx), rows, :], so[b])
            pltpu.async_copy(sh_head, out_hbm.at[pl.ds(0, HEAD), rows, :],
                             sconst)
            pltpu.async_copy(sh_tail,
                             out_hbm.at[pl.ds(COMB + n_ctx, TAIL), rows, :],
                             sconst)

        def wait_out(b):
            rows = pl.ds(0, CHUNK)
            pltpu.make_async_copy(
                comb[b], out_hbm.at[pl.ds(COMB, n_ctx), rows, :],
                so[b]).wait()

        def wait_const():
            rows = pl.ds(0, CHUNK)
            pltpu.make_async_copy(
                sh_head, out_hbm.at[pl.ds(0, HEAD), rows, :], sconst).wait()
            pltpu.make_async_copy(
                sh_tail, out_hbm.at[pl.ds(COMB + n_ctx, TAIL), rows, :],
                sconst).wait()

        def compute(b):
            gb = gath[b]
            cb = comb[b]
            for cc in range(n_ctx):
                @plsc.parallel_loop(0, D // L, unroll=8)
                def _cj(j):
                    s = j * L
                    cvec = ctx_v[cc, pl.ds(s, L)]
                    for r in range(CHUNK):
                        cb[cc, r, pl.ds(s, L)] = gb[r, cc, pl.ds(s, L)] + cvec

        start_gather(0, 0)
        start_gather(1, 1)

        def pair_body(i, _):
            for par in range(2):
                c = 2 * i + par
                wait_gather(par)

                @pl.when(i > 0)
                def _wo():
                    wait_out(par)

                compute(par)

                @pl.when(c + 2 < NCHUNK)
                def _sg():
                    start_gather(c + 2, par)

                start_out(c, par)
            return ()

        lax.fori_loop(0, NCHUNK // 2, pair_body, (), unroll=False)
        wait_out(0)
        wait_out(1)
        for _ in range(NCHUNK):
            wait_const()

    return k


def kernel(pids, ctx, class_ctx, prefix_emb, suffix_emb, sos_emb, eos_emb):
    B = pids.shape[0]
    n_ctx, D = ctx.shape
    V = class_ctx.shape[0]
    pre_len = prefix_emb.shape[0]
    suf_len = suffix_emb.shape[0]

    k = _build(B, D, n_ctx, pre_len, suf_len, V)
    out = k(pids.astype(jnp.int32), ctx, class_ctx, prefix_emb, suffix_emb,
            sos_emb, eos_emb)
    return out.transpose(1, 0, 2)


# speedup vs baseline: 1.4670x; 1.0133x over previous
import functools

import jax
import jax.numpy as jnp
from jax import lax
from jax.experimental import pallas as pl
from jax.experimental.pallas import tpu as pltpu
from jax.experimental.pallas import tpu_sc as plsc

L = 16
NC = 2
NS = 16
NW = NC * NS


def _build_sc(B, D, n_ctx, pre_len, suf_len, V):
    SEQ = 1 + pre_len + n_ctx + suf_len + 1
    COMB = 1 + pre_len
    BPW = B // NW
    CHUNK = 8
    NCHUNK = BPW // CHUNK

    mesh = plsc.VectorSubcoreMesh(core_axis_name="c", subcore_axis_name="s")

    @functools.partial(
        pl.kernel,
        out_type=jax.ShapeDtypeStruct((SEQ, B, D), jnp.float32),
        mesh=mesh,
        scratch_types=[
            pltpu.VMEM((BPW,), jnp.int32),
            pltpu.VMEM((n_ctx, D), jnp.float32),
            pltpu.VMEM((CHUNK, n_ctx, D), jnp.float32),
            pltpu.VMEM((CHUNK, n_ctx, D), jnp.float32),
            pltpu.VMEM((n_ctx, CHUNK, D), jnp.float32),
            pltpu.VMEM((n_ctx, CHUNK, D), jnp.float32),
            pltpu.SemaphoreType.DMA,
            pltpu.SemaphoreType.DMA,
            pltpu.SemaphoreType.DMA,
            pltpu.SemaphoreType.DMA,
        ],
    )
    def k(pids_hbm, ctx_hbm, table_hbm, out_hbm, idx_v, ctx_v,
          gath0, gath1, comb0, comb1, sg0, sg1, so0, so1):
        gath = (gath0, gath1)
        comb = (comb0, comb1)
        sg = (sg0, sg1)
        so = (so0, so1)
        wid = lax.axis_index("s") * NC + lax.axis_index("c")
        base = wid * BPW
        pltpu.sync_copy(pids_hbm.at[pl.ds(base, BPW)], idx_v)
        pltpu.sync_copy(ctx_hbm, ctx_v)

        def start_gather(c, b):
            pltpu.async_copy(
                table_hbm.at[idx_v.at[pl.ds(c * CHUNK, CHUNK)]],
                gath[b], sg[b])

        def wait_gather(b):
            pltpu.make_async_copy(
                table_hbm.at[pl.ds(0, CHUNK)], gath[b], sg[b]).wait()

        def start_out(c, b):
            rows = pl.ds(base + c * CHUNK, CHUNK)
            pltpu.async_copy(comb[b],
                             out_hbm.at[pl.ds(COMB, n_ctx), rows, :], so[b])

        def wait_out(b):
            rows = pl.ds(0, CHUNK)
            pltpu.make_async_copy(
                comb[b], out_hbm.at[pl.ds(COMB, n_ctx), rows, :],
                so[b]).wait()

        def compute(b):
            gb = gath[b]
            cb = comb[b]
            for cc in range(n_ctx):
                @plsc.parallel_loop(0, D // L, unroll=8)
                def _cj(j):
                    s = j * L
                    cvec = ctx_v[cc, pl.ds(s, L)]
                    for r in range(CHUNK):
                        cb[cc, r, pl.ds(s, L)] = gb[r, cc, pl.ds(s, L)] + cvec

        start_gather(0, 0)
        start_gather(1, 1)

        def pair_body(i, _):
            for par in range(2):
                c = 2 * i + par
                wait_gather(par)

                @pl.when(i > 0)
                def _wo():
                    wait_out(par)

                compute(par)

                @pl.when(c + 2 < NCHUNK)
                def _sg():
                    start_gather(c + 2, par)

                start_out(c, par)
            return ()

        lax.fori_loop(0, NCHUNK // 2, pair_body, (), unroll=False)
        wait_out(0)
        wait_out(1)

    return k


def _build_const_fill(B, D, SEQ, HEAD):
    RB = 1024
    NP = HEAD + 2

    def body(const_ref, big_ref, out_ref):
        del big_ref
        row = const_ref[0, 0]
        out_ref[...] = jnp.broadcast_to(row[None, None, :], (1, RB, D))

    def idx_out(g, nb):
        return (jnp.where(g >= HEAD, g + (SEQ - NP), g), nb, 0)

    return pl.pallas_call(
        body,
        grid=(NP, B // RB),
        in_specs=[
            pl.BlockSpec((1, 1, D), lambda g, nb: (g, 0, 0)),
            pl.BlockSpec(memory_space=pl.ANY),
        ],
        out_specs=pl.BlockSpec((1, RB, D), idx_out),
        out_shape=jax.ShapeDtypeStruct((SEQ, B, D), jnp.float32),
        input_output_aliases={1: 0},
    )


def kernel(pids, ctx, class_ctx, prefix_emb, suffix_emb, sos_emb, eos_emb):
    B = pids.shape[0]
    n_ctx, D = ctx.shape
    V = class_ctx.shape[0]
    pre_len = prefix_emb.shape[0]
    suf_len = suffix_emb.shape[0]
    SEQ = 1 + pre_len + n_ctx + suf_len + 1
    HEAD = 1 + pre_len

    sc = _build_sc(B, D, n_ctx, pre_len, suf_len, V)
    out = sc(pids.astype(jnp.int32), ctx, class_ctx)
    consts = jnp.concatenate([sos_emb, prefix_emb, suffix_emb, eos_emb],
                             axis=0)[:, None, :]
    out = _build_const_fill(B, D, SEQ, HEAD)(consts, out)
    return out.transpose(1, 0, 2)


# speedup vs baseline: 1.5252x; 1.0397x over previous
import functools

import jax
import jax.numpy as jnp
from jax import lax
from jax.experimental import pallas as pl
from jax.experimental.pallas import tpu as pltpu
from jax.experimental.pallas import tpu_sc as plsc

L = 16
NC = 2
NS = 16
NW = NC * NS


def _build_sc(B, D, n_ctx, pre_len, suf_len, V):
    SEQ = 1 + pre_len + n_ctx + suf_len + 1
    COMB = 1 + pre_len
    BPW = B // NW
    CHUNK = 8
    NCHUNK = BPW // CHUNK

    mesh = plsc.VectorSubcoreMesh(core_axis_name="c", subcore_axis_name="s")

    @functools.partial(
        pl.kernel,
        out_type=jax.ShapeDtypeStruct((SEQ, B, D), jnp.float32),
        mesh=mesh,
        scratch_types=[
            pltpu.VMEM((BPW,), jnp.int32),
            pltpu.VMEM((n_ctx, D), jnp.float32),
            pltpu.VMEM((CHUNK, n_ctx, D), jnp.float32),
            pltpu.VMEM((CHUNK, n_ctx, D), jnp.float32),
            pltpu.VMEM((n_ctx, CHUNK, D), jnp.float32),
            pltpu.VMEM((n_ctx, CHUNK, D), jnp.float32),
            pltpu.SemaphoreType.DMA,
            pltpu.SemaphoreType.DMA,
            pltpu.SemaphoreType.DMA,
            pltpu.SemaphoreType.DMA,
        ],
    )
    def k(pids_hbm, ctx_hbm, table_hbm, out_hbm, idx_v, ctx_v,
          gath0, gath1, comb0, comb1, sg0, sg1, so0, so1):
        gath = (gath0, gath1)
        comb = (comb0, comb1)
        sg = (sg0, sg1)
        so = (so0, so1)
        wid = lax.axis_index("s") * NC + lax.axis_index("c")
        base = wid * BPW
        pltpu.sync_copy(pids_hbm.at[pl.ds(base, BPW)], idx_v)
        pltpu.sync_copy(ctx_hbm, ctx_v)

        def start_gather(c, b):
            pltpu.async_copy(
                table_hbm.at[idx_v.at[pl.ds(c * CHUNK, CHUNK)]],
                gath[b], sg[b])

        def wait_gather(b):
            pltpu.make_async_copy(
                table_hbm.at[pl.ds(0, CHUNK)], gath[b], sg[b]).wait()

        def start_out(c, b):
            rows = pl.ds(base + c * CHUNK, CHUNK)
            pltpu.async_copy(comb[b],
                             out_hbm.at[pl.ds(COMB, n_ctx), rows, :], so[b])

        def wait_out(b):
            rows = pl.ds(0, CHUNK)
            pltpu.make_async_copy(
                comb[b], out_hbm.at[pl.ds(COMB, n_ctx), rows, :],
                so[b]).wait()

        def compute(b):
            gb = gath[b]
            cb = comb[b]
            for cc in range(n_ctx):
                @plsc.parallel_loop(0, D // L, unroll=8)
                def _cj(j):
                    s = j * L
                    cvec = ctx_v[cc, pl.ds(s, L)]
                    for r in range(CHUNK):
                        cb[cc, r, pl.ds(s, L)] = gb[r, cc, pl.ds(s, L)] + cvec

        start_gather(0, 0)
        start_gather(1, 1)

        def pair_body(i, _):
            for par in range(2):
                c = 2 * i + par
                wait_gather(par)

                @pl.when(i > 0)
                def _wo():
                    wait_out(par)

                compute(par)

                @pl.when(c + 2 < NCHUNK)
                def _sg():
                    start_gather(c + 2, par)

                start_out(c, par)
            return ()

        lax.fori_loop(0, NCHUNK // 2, pair_body, (), unroll=False)
        wait_out(0)
        wait_out(1)

    return k


def _build_const_fill(B, D, SEQ, HEAD):
    RB = 4096
    NP = HEAD + 2

    def body(const_ref, big_ref, out_ref):
        del big_ref
        row = const_ref[0, 0]
        out_ref[...] = jnp.broadcast_to(row[None, None, :], (1, RB, D))

    def idx_out(g, nb):
        return (jnp.where(g >= HEAD, g + (SEQ - NP), g), nb, 0)

    return pl.pallas_call(
        body,
        grid=(NP, B // RB),
        in_specs=[
            pl.BlockSpec((1, 1, D), lambda g, nb: (g, 0, 0)),
            pl.BlockSpec(memory_space=pl.ANY),
        ],
        out_specs=pl.BlockSpec((1, RB, D), idx_out),
        out_shape=jax.ShapeDtypeStruct((SEQ, B, D), jnp.float32),
        input_output_aliases={1: 0},
    )


def kernel(pids, ctx, class_ctx, prefix_emb, suffix_emb, sos_emb, eos_emb):
    B = pids.shape[0]
    n_ctx, D = ctx.shape
    V = class_ctx.shape[0]
    pre_len = prefix_emb.shape[0]
    suf_len = suffix_emb.shape[0]
    SEQ = 1 + pre_len + n_ctx + suf_len + 1
    HEAD = 1 + pre_len

    sc = _build_sc(B, D, n_ctx, pre_len, suf_len, V)
    out = sc(pids.astype(jnp.int32), ctx, class_ctx)
    consts = jnp.concatenate([sos_emb, prefix_emb, suffix_emb, eos_emb],
                             axis=0)[:, None, :]
    out = _build_const_fill(B, D, SEQ, HEAD)(consts, out)
    return out.transpose(1, 0, 2)
